# Initial kernel scaffold; baseline (speedup 1.0000x reference)
import jax
import jax.numpy as jnp
from jax.experimental import pallas as pl


def kernel(hashes, indices, tok_emb, hash_proj, bridge_W, bridge_b, log_c, pos_tan, Wq, bq, Wk, bk, Wv, bv, Wo, bo, Wf1, bf1, Wf2, bf2, g1, be1, g2, be2, log_tau, out_W, out_b):
    raise NotImplementedError("write your pallas kernel here")


# speedup vs baseline: 7.2755x; 7.2755x over previous
import math

import jax
import jax.numpy as jnp
from jax.experimental import pallas as pl
from jax.experimental.pallas import tpu as pltpu

N = 2048
D = 768
H = 12
HD = 64
KNN = 16
VOCAB = 256
MOD = 2147483647.0
BLK = 256
NBLK = N // BLK
NEG = -1e30

_CP = pltpu.CompilerParams(
    dimension_semantics=("arbitrary",),
    vmem_limit_bytes=100 * 1024 * 1024,
)


def _blk_spec(shape):
    return pl.BlockSpec(shape, lambda i: (i,) + (0,) * (len(shape) - 1))


def _const_spec(shape):
    return pl.BlockSpec(shape, lambda i: (0,) * len(shape))


def _embed_kernel(idx_ref, hf_ref, temb_ref, hp_ref, w1_ref, w2_ref, b_ref,
                  o_ref):
    idx = idx_ref[...]
    cols = jax.lax.broadcasted_iota(jnp.int32, (BLK, VOCAB), 1)
    onehot = jnp.where(idx == cols, 1.0, 0.0)
    ce = jnp.dot(onehot, temb_ref[...], preferred_element_type=jnp.float32)
    x = jnp.dot(ce, w1_ref[...], preferred_element_type=jnp.float32)
    hp2 = jnp.dot(hp_ref[...], w2_ref[...], preferred_element_type=jnp.float32)
    h = hf_ref[...] / MOD
    o_ref[...] = x + h * hp2 + b_ref[...]


def _expmap_kernel(pt_ref, lc_ref, o_ref):
    v = pt_ref[...]
    c = jnp.exp(lc_ref[0, 0])
    sqrt_c = jnp.sqrt(c)
    vn = jnp.sqrt(jnp.sum(v * v, axis=1, keepdims=True))
    vn = jnp.maximum(vn, 1e-8)
    o_ref[...] = jnp.tanh(sqrt_c * vn) / (sqrt_c * vn) * v


def _bias_kernel(lc_ref, pos_ref, o_ref):
    i = pl.program_id(0)
    c = jnp.exp(lc_ref[0, 0])
    p = pos_ref[...]
    pb = pos_ref[pl.ds(i * BLK, BLK), :]
    g = jax.lax.dot_general(pb, p, (((1,), (1,)), ((), ())),
                            preferred_element_type=jnp.float32)
    sqb = jnp.sum(pb * pb, axis=1, keepdims=True)
    ones = jnp.ones((1, D), jnp.float32)
    sqf = jax.lax.dot_general(ones, p * p, (((1,), (1,)), ((), ())),
                              preferred_element_type=jnp.float32)
    d2 = jnp.maximum(sqb + sqf - 2.0 * g, 0.0)
    denom = (1.0 - c * sqb) * (1.0 - c * sqf) + 1e-8
    arg = jnp.maximum(1.0 + 2.0 * c * d2 / denom, 1.0)
    dist = jnp.log(arg + jnp.sqrt((arg + 1.0) * (arg - 1.0))) / jnp.sqrt(c)
    negd = -dist
    work = negd
    sel = jnp.zeros((BLK, N), jnp.bool_)
    col = jax.lax.broadcasted_iota(jnp.int32, (BLK, N), 1)
    for _ in range(KNN):
        m = jnp.max(work, axis=1, keepdims=True)
        first = jnp.min(jnp.where(work == m, col, N), axis=1, keepdims=True)
        hit = col == first
        sel = jnp.logical_or(sel, hit)
        work = jnp.where(hit, -jnp.inf, work)
    o_ref[...] = jnp.where(sel, negd, NEG)


def _layernorm(x, g, b):
    m = jnp.mean(x, axis=1, keepdims=True)
    v = jnp.mean((x - m) * (x - m), axis=1, keepdims=True)
    return (x - m) / jnp.sqrt(v + 1e-5) * g + b


def _qkv_kernel(x_ref, g_ref, b_ref, wq_ref, bq_ref, wk_ref, bk_ref, wv_ref,
                bv_ref, q_ref, k_ref, v_ref):
    xn = _layernorm(x_ref[...], g_ref[...], b_ref[...])
    q_ref[...] = jnp.dot(xn, wq_ref[...],
                         preferred_element_type=jnp.float32) + bq_ref[...]
    k_ref[...] = jnp.dot(xn, wk_ref[...],
                         preferred_element_type=jnp.float32) + bk_ref[...]
    v_ref[...] = jnp.dot(xn, wv_ref[...],
                         preferred_element_type=jnp.float32) + bv_ref[...]


def _attn_kernel(lt_ref, q_ref, k_ref, v_ref, bias_ref, o_ref):
    tau = jnp.exp(lt_ref[0, 0]) + 1e-8
    bias = bias_ref[...] / tau
    scale = 1.0 / math.sqrt(HD)
    for h in range(H):
        sl = slice(h * HD, (h + 1) * HD)
        qh = q_ref[:, sl]
        kh = k_ref[:, sl]
        vh = v_ref[:, sl]
        s = jax.lax.dot_general(qh, kh, (((1,), (1,)), ((), ())),
                                preferred_element_type=jnp.float32)
        s = s * scale + bias
        m = jnp.max(s, axis=1, keepdims=True)
        e = jnp.exp(s - m)
        p = e / jnp.sum(e, axis=1, keepdims=True)
        o_ref[:, sl] = jnp.dot(p, vh, preferred_element_type=jnp.float32)


def _ffn_body(x_ref, ao_ref, wo_ref, bo_ref, g_ref, b_ref, wf1_ref, bf1_ref,
              wf2_ref, bf2_ref):
    x1 = x_ref[...] + jnp.dot(ao_ref[...], wo_ref[...],
                              preferred_element_type=jnp.float32) + bo_ref[...]
    x2 = _layernorm(x1, g_ref[...], b_ref[...])
    pre = jnp.dot(x2, wf1_ref[...],
                  preferred_element_type=jnp.float32) + bf1_ref[...]
    mid = 0.5 * pre * (1.0 + jax.lax.erf(pre * (1.0 / math.sqrt(2.0))))
    return x1 + jnp.dot(mid, wf2_ref[...],
                        preferred_element_type=jnp.float32) + bf2_ref[...]


def _post_kernel(x_ref, ao_ref, wo_ref, bo_ref, g_ref, b_ref, wf1_ref,
                 bf1_ref, wf2_ref, bf2_ref, o_ref):
    o_ref[...] = _ffn_body(x_ref, ao_ref, wo_ref, bo_ref, g_ref, b_ref,
                           wf1_ref, bf1_ref, wf2_ref, bf2_ref)


def _post_final_kernel(x_ref, ao_ref, wo_ref, bo_ref, g_ref, b_ref, wf1_ref,
                       bf1_ref, wf2_ref, bf2_ref, ow_ref, ob_ref, o_ref,
                       logit_ref):
    xo = _ffn_body(x_ref, ao_ref, wo_ref, bo_ref, g_ref, b_ref, wf1_ref,
                   bf1_ref, wf2_ref, bf2_ref)
    o_ref[...] = xo
    logit_ref[...] = jnp.dot(xo[BLK - 1:BLK, :], ow_ref[...],
                             preferred_element_type=jnp.float32) + ob_ref[...]


def kernel(hashes, indices, tok_emb, hash_proj, bridge_W, bridge_b, log_c,
           pos_tan, Wq, bq, Wk, bk, Wv, bv, Wo, bo, Wf1, bf1, Wf2, bf2, g1,
           be1, g2, be2, log_tau, out_W, out_b):
    f32 = jnp.float32
    idxc = indices.reshape(N, 1).astype(jnp.int32)
    hf = hashes.reshape(N, 1).astype(f32)
    lc = log_c.reshape(1, 1).astype(f32)

    x = pl.pallas_call(
        _embed_kernel,
        grid=(NBLK,),
        in_specs=[
            _blk_spec((BLK, 1)),
            _blk_spec((BLK, 1)),
            _const_spec((VOCAB, D)),
            _const_spec((1, D)),
            _const_spec((D, D)),
            _const_spec((D, D)),
            _const_spec((1, D)),
        ],
        out_specs=_blk_spec((BLK, D)),
        out_shape=jax.ShapeDtypeStruct((N, D), f32),
        compiler_params=_CP,
    )(idxc, hf, tok_emb, hash_proj, bridge_W[:D], bridge_W[D:],
      bridge_b.reshape(1, D))

    pos = pl.pallas_call(
        _expmap_kernel,
        grid=(NBLK,),
        in_specs=[_blk_spec((BLK, D)), _const_spec((1, 1))],
        out_specs=_blk_spec((BLK, D)),
        out_shape=jax.ShapeDtypeStruct((N, D), f32),
        compiler_params=_CP,
    )(pos_tan, lc)

    nbias = pl.pallas_call(
        _bias_kernel,
        grid=(NBLK,),
        in_specs=[_const_spec((1, 1)), _const_spec((N, D))],
        out_specs=_blk_spec((BLK, N)),
        out_shape=jax.ShapeDtypeStruct((N, N), f32),
        compiler_params=_CP,
    )(lc, pos)

    logits = None
    for i in range(2):
        q, k, v = pl.pallas_call(
            _qkv_kernel,
            grid=(NBLK,),
            in_specs=[
                _blk_spec((BLK, D)),
                _const_spec((1, D)),
                _const_spec((1, D)),
                _const_spec((D, D)),
                _const_spec((1, D)),
                _const_spec((D, D)),
                _const_spec((1, D)),
                _const_spec((D, D)),
                _const_spec((1, D)),
            ],
            out_specs=[_blk_spec((BLK, D))] * 3,
            out_shape=[jax.ShapeDtypeStruct((N, D), f32)] * 3,
            compiler_params=_CP,
        )(x, g1[i].reshape(1, D), be1[i].reshape(1, D), Wq[i],
          bq[i].reshape(1, D), Wk[i], bk[i].reshape(1, D), Wv[i],
          bv[i].reshape(1, D))

        ao = pl.pallas_call(
            _attn_kernel,
            grid=(NBLK,),
            in_specs=[
                _const_spec((1, 1)),
                _blk_spec((BLK, D)),
                _const_spec((N, D)),
                _const_spec((N, D)),
                _blk_spec((BLK, N)),
            ],
            out_specs=_blk_spec((BLK, D)),
            out_shape=jax.ShapeDtypeStruct((N, D), f32),
            compiler_params=_CP,
        )(log_tau[i].reshape(1, 1), q, k, v, nbias)

        post_in = (x, ao, Wo[i], bo[i].reshape(1, D), g2[i].reshape(1, D),
                   be2[i].reshape(1, D), Wf1[i], bf1[i].reshape(1, 4 * D),
                   Wf2[i], bf2[i].reshape(1, D))
        post_specs = [
            _blk_spec((BLK, D)),
            _blk_spec((BLK, D)),
            _const_spec((D, D)),
            _const_spec((1, D)),
            _const_spec((1, D)),
            _const_spec((1, D)),
            _const_spec((D, 4 * D)),
            _const_spec((1, 4 * D)),
            _const_spec((4 * D, D)),
            _const_spec((1, D)),
        ]
        if i == 0:
            x = pl.pallas_call(
                _post_kernel,
                grid=(NBLK,),
                in_specs=post_specs,
                out_specs=_blk_spec((BLK, D)),
                out_shape=jax.ShapeDtypeStruct((N, D), f32),
                compiler_params=_CP,
            )(*post_in)
        else:
            x, logits = pl.pallas_call(
                _post_final_kernel,
                grid=(NBLK,),
                in_specs=post_specs + [
                    _const_spec((D, VOCAB)),
                    _const_spec((1, VOCAB)),
                ],
                out_specs=[_blk_spec((BLK, D)), _const_spec((1, VOCAB))],
                out_shape=[
                    jax.ShapeDtypeStruct((N, D), f32),
                    jax.ShapeDtypeStruct((1, VOCAB), f32),
                ],
                compiler_params=_CP,
            )(*post_in, out_W, out_b.reshape(1, VOCAB))
    return logits
